# Initial kernel scaffold; baseline (speedup 1.0000x reference)
import jax
import jax.numpy as jnp
from jax.experimental import pallas as pl


def kernel(input, W0, W1):
    raise NotImplementedError("write your pallas kernel here")


# speedup vs baseline: 2.1166x; 2.1166x over previous
import functools

import jax
import jax.numpy as jnp
from jax import lax
from jax.experimental import pallas as pl
from jax.experimental.pallas import tpu as pltpu
from jax.experimental.pallas import tpu_sc as plsc

SPLIT = 500_000
D = 32
L = 16
NC, NS = 2, 16
NW = NC * NS

CHUNK = 512
G = 128


def _sc_body(idx_hbm, w0_hbm, w1_hbm, out_hbm,
             idx_v, idx1_v, rows0_v, rows1_v, sem0, sem1):
    n_total = idx_hbm.shape[0]
    per_w = n_total // NW
    wid = lax.axis_index("s") * NC + lax.axis_index("c")
    base_w = wid * per_w

    @pl.loop(0, per_w // CHUNK)
    def _chunk(g):
        base = base_w + g * CHUNK
        pltpu.sync_copy(idx_hbm.at[pl.ds(base, CHUNK)], idx_v)

        @pl.loop(0, CHUNK // L)
        def _remap(i):
            v = idx_v[pl.ds(i * L, L)]
            idx1_v[pl.ds(i * L, L)] = jnp.where(v >= SPLIT, v - SPLIT, v)

        cps = []
        for j in range(CHUNK // G):
            cps.append(pltpu.async_copy(
                w0_hbm.at[idx_v.at[pl.ds(j * G, G)]],
                rows0_v.at[pl.ds(j * G, G)], sem0))
            cps.append(pltpu.async_copy(
                w1_hbm.at[idx1_v.at[pl.ds(j * G, G)]],
                rows1_v.at[pl.ds(j * G, G)], sem1))
        for cp in cps:
            cp.wait()

        @pl.loop(0, CHUNK, step=L)
        def _mask(r0):
            v = idx_v[pl.ds(r0, L)]
            sf = jnp.where(v >= SPLIT, 1.0, 0.0)
            for rr in range(L):
                r = r0 + rr
                a = rows1_v[r, pl.ds(0, L)]
                b = rows1_v[r, pl.ds(L, L)]
                rows1_v[r, pl.ds(0, L)] = a * sf[rr]
                rows1_v[r, pl.ds(L, L)] = b * sf[rr]

        pltpu.sync_copy(rows0_v, out_hbm.at[pl.ds(base, CHUNK), pl.ds(0, D)])
        pltpu.sync_copy(rows1_v, out_hbm.at[pl.ds(base, CHUNK), pl.ds(D, D)])


@jax.jit
def kernel(input, W0, W1):
    n = input.shape[0] * input.shape[1]
    idx = input.reshape(n).astype(jnp.int32)

    mesh = plsc.VectorSubcoreMesh(
        core_axis_name="c", subcore_axis_name="s",
        num_cores=NC, num_subcores=NS)
    out = pl.kernel(
        _sc_body,
        out_type=jax.ShapeDtypeStruct((n, 2 * D), jnp.float32),
        mesh=mesh,
        compiler_params=pltpu.CompilerParams(use_tc_tiling_on_sc=False),
        scratch_types=[
            pltpu.VMEM((CHUNK,), jnp.int32),
            pltpu.VMEM((CHUNK,), jnp.int32),
            pltpu.VMEM((CHUNK, D), jnp.float32),
            pltpu.VMEM((CHUNK, D), jnp.float32),
            pltpu.SemaphoreType.DMA,
            pltpu.SemaphoreType.DMA,
        ],
    )(idx, W0, W1)
    return out.reshape(input.shape[0], input.shape[1], 2 * D)
